# baseline (speedup 1.0000x reference)
import functools

import jax
import jax.numpy as jnp
from jax import lax
from jax.experimental import pallas as pl
from jax.experimental.pallas import tpu as pltpu
from jax.experimental.pallas import tpu_sc as plsc

EMBED_DIM = 64
NC = 2
NS = 16
NW = NC * NS
CHUNK = 128
NBUF = 6


@functools.lru_cache(maxsize=None)
def _build_gather(total_rows):
    rows_per_w = total_rows // NW
    n_chunks = rows_per_w // CHUNK
    mesh = plsc.VectorSubcoreMesh(core_axis_name="c", subcore_axis_name="s")

    idx_rows = (n_chunks + 14) // 8 * 8

    @functools.partial(
        pl.kernel,
        out_type=jax.ShapeDtypeStruct((total_rows, 2 * EMBED_DIM),
                                      jnp.float32),
        mesh=mesh,
        compiler_params=pltpu.CompilerParams(use_tc_tiling_on_sc=True),
        scratch_types=[
            pltpu.VMEM((idx_rows, CHUNK), jnp.int32),
        ] + [pltpu.VMEM((CHUNK, 2 * EMBED_DIM), jnp.float32)] * NBUF
          + [pltpu.SemaphoreType.DMA] * (2 * NBUF),
    )
    def gather_kernel(idx_hbm, table_hbm, out_hbm, idx_v, *rest):
        bufs = rest[:NBUF]
        gsems = rest[NBUF:2 * NBUF]
        ssems = rest[2 * NBUF:]
        wid = lax.axis_index("s") * NC + lax.axis_index("c")
        base = pl.multiple_of(wid * rows_per_w, CHUNK)
        start = wid * n_chunks
        start8 = pl.multiple_of((start // 8) * 8, 8)
        off = start - start8
        pltpu.sync_copy(idx_hbm.at[pl.ds(start8, idx_rows)], idx_v)

        gathers = [None] * NBUF
        scatters = [None] * NBUF
        for i in range(min(NBUF - 1, n_chunks)):
            gathers[i] = pltpu.async_copy(
                table_hbm.at[idx_v.at[off + i]], bufs[i], gsems[i])
        for i in range(n_chunks):
            b = i % NBUF
            j = i + NBUF - 1
            if j < n_chunks:
                jb = j % NBUF
                if scatters[jb] is not None:
                    scatters[jb].wait()
                    scatters[jb] = None
                gathers[jb] = pltpu.async_copy(
                    table_hbm.at[idx_v.at[off + j]], bufs[jb], gsems[jb])
            gathers[b].wait()
            scatters[b] = pltpu.async_copy(
                bufs[b], out_hbm.at[pl.ds(base + i * CHUNK, CHUNK)],
                ssems[b])
        for s in scatters:
            if s is not None:
                s.wait()

    return gather_kernel


def kernel(c, img, q, cap_len, table):
    batch, cap_len_dim = c.shape
    total_rows = batch * cap_len_dim
    idx = c.T.reshape(total_rows // CHUNK, CHUNK).astype(jnp.int32)
    table_pad = jnp.concatenate([table, table], axis=1)
    flat = _build_gather(total_rows)(idx, table_pad)
    c_emb = (flat.reshape(cap_len_dim, batch, 2 * EMBED_DIM)[:, :, :EMBED_DIM]
             .transpose(1, 0, 2))
    return (img, c_emb, c, cap_len)


# speedup vs baseline: 1.0824x; 1.0824x over previous
import functools

import jax
import jax.numpy as jnp
from jax import lax
from jax.experimental import pallas as pl
from jax.experimental.pallas import tpu as pltpu
from jax.experimental.pallas import tpu_sc as plsc

EMBED_DIM = 64
NC = 2
NS = 16
NW = NC * NS
CHUNK = 128
NBUF = 7


@functools.lru_cache(maxsize=None)
def _build_gather(total_rows):
    rows_per_w = total_rows // NW
    n_chunks = rows_per_w // CHUNK
    mesh = plsc.VectorSubcoreMesh(core_axis_name="c", subcore_axis_name="s")

    idx_rows = (n_chunks + 14) // 8 * 8

    @functools.partial(
        pl.kernel,
        out_type=jax.ShapeDtypeStruct((total_rows, 2 * EMBED_DIM),
                                      jnp.float32),
        mesh=mesh,
        compiler_params=pltpu.CompilerParams(use_tc_tiling_on_sc=True),
        scratch_types=[
            pltpu.VMEM((idx_rows, CHUNK), jnp.int32),
        ] + [pltpu.VMEM((CHUNK, 2 * EMBED_DIM), jnp.float32)] * NBUF
          + [pltpu.SemaphoreType.DMA] * (2 * NBUF),
    )
    def gather_kernel(idx_hbm, table_hbm, out_hbm, idx_v, *rest):
        bufs = rest[:NBUF]
        gsems = rest[NBUF:2 * NBUF]
        ssems = rest[2 * NBUF:]
        wid = lax.axis_index("s") * NC + lax.axis_index("c")
        base = pl.multiple_of(wid * rows_per_w, CHUNK)
        start = wid * n_chunks
        start8 = pl.multiple_of((start // 8) * 8, 8)
        off = start - start8
        pltpu.sync_copy(idx_hbm.at[pl.ds(start8, idx_rows)], idx_v)

        gathers = [None] * NBUF
        scatters = [None] * NBUF
        for i in range(min(NBUF - 1, n_chunks)):
            gathers[i] = pltpu.async_copy(
                table_hbm.at[idx_v.at[off + i]], bufs[i], gsems[i])
        for i in range(n_chunks):
            b = i % NBUF
            j = i + NBUF - 1
            if j < n_chunks:
                jb = j % NBUF
                if scatters[jb] is not None:
                    scatters[jb].wait()
                    scatters[jb] = None
                gathers[jb] = pltpu.async_copy(
                    table_hbm.at[idx_v.at[off + j]], bufs[jb], gsems[jb])
            gathers[b].wait()
            scatters[b] = pltpu.async_copy(
                bufs[b], out_hbm.at[pl.ds(base + i * CHUNK, CHUNK)],
                ssems[b])
        for s in scatters:
            if s is not None:
                s.wait()

    return gather_kernel


def kernel(c, img, q, cap_len, table):
    batch, cap_len_dim = c.shape
    total_rows = batch * cap_len_dim
    idx = c.T.reshape(total_rows // CHUNK, CHUNK).astype(jnp.int32)
    table_pad = jnp.pad(table, ((0, 0), (0, EMBED_DIM)))
    flat = _build_gather(total_rows)(idx, table_pad)
    c_emb = (flat.reshape(cap_len_dim, batch, 2 * EMBED_DIM)[:, :, :EMBED_DIM]
             .transpose(1, 0, 2))
    return (img, c_emb, c, cap_len)
